# baseline (speedup 1.0000x reference)
import functools

import jax
import jax.numpy as jnp
from jax import lax
from jax.experimental import pallas as pl
from jax.experimental.pallas import tpu as pltpu
from jax.experimental.pallas import tpu_sc as plsc

S = 512
D = 512
NC = 2
NS = 16
NW = NC * NS
ROWS_STAGE = 1024 // NS
ROWS_OUT = S // NW

_mesh = plsc.VectorSubcoreMesh(core_axis_name="c", subcore_axis_name="s")


@functools.partial(
    pl.kernel,
    mesh=_mesh,
    out_type=jax.ShapeDtypeStruct((S * S * D,), jnp.float32),
    scratch_types=[
        pltpu.VMEM_SHARED((2 * S * D,), jnp.float32),
        pltpu.SemaphoreType.DMA,
        pltpu.SemaphoreType.DMA,
    ],
)
def _expand(relp_hbm, out_hbm, rev_sh, sem_a, sem_b):
    c = lax.axis_index("c")
    s = lax.axis_index("s")
    wid = s * NC + c

    kb = s * ROWS_STAGE
    for r in range(ROWS_STAGE):
        k = kb + r
        pltpu.make_async_copy(
            relp_hbm.at[pl.ds(((2 * S - 1) - k) * D, D)],
            rev_sh.at[pl.ds(k * D, D)], sem_a).start()
    for _ in range(ROWS_STAGE):
        pltpu.make_async_copy(
            relp_hbm.at[pl.ds(0, D)], rev_sh.at[pl.ds(0, D)], sem_a).wait()

    plsc.subcore_barrier()

    ib = wid * ROWS_OUT
    for r in range(ROWS_OUT):
        i = ib + r
        pltpu.make_async_copy(
            rev_sh.at[pl.ds(((S - 1) - i) * D, S * D)],
            out_hbm.at[pl.ds(i * S * D, S * D)], sem_b).start()
    for _ in range(ROWS_OUT):
        pltpu.make_async_copy(
            rev_sh.at[pl.ds(0, S * D)],
            out_hbm.at[pl.ds(0, S * D)], sem_b).wait()


def kernel(seq_len, rel_pos_emb):
    del seq_len
    max_len = (rel_pos_emb.shape[0] + 1) // 2
    start = max_len - 1 - (S - 1)
    relp = lax.slice_in_dim(rel_pos_emb, start - 1, start + 2 * S - 1, axis=0)
    out_flat = _expand(relp.reshape(2 * S * D))
    return out_flat.reshape(S, S, D)


# speedup vs baseline: 1.1374x; 1.1374x over previous
import functools

import jax
import jax.numpy as jnp
from jax import lax
from jax.experimental import pallas as pl
from jax.experimental.pallas import tpu as pltpu
from jax.experimental.pallas import tpu_sc as plsc

S = 512
D = 512
NC = 2
NS = 16
NW = NC * NS
NI = 8
NJ = 4
BI = S // NI
BJ = S // NJ
WROWS = BI + BJ - 1

_mesh = plsc.VectorSubcoreMesh(core_axis_name="c", subcore_axis_name="s")


@functools.partial(
    pl.kernel,
    mesh=_mesh,
    out_type=jax.ShapeDtypeStruct((S * S * D,), jnp.float32),
    scratch_types=[
        pltpu.VMEM((WROWS * D,), jnp.float32),
        pltpu.SemaphoreType.DMA,
        pltpu.SemaphoreType.DMA,
    ],
)
def _expand(relp_hbm, out_hbm, win_v, sem_r, sem_w):
    wid = lax.axis_index("s") * NC + lax.axis_index("c")
    i0 = (wid // NJ) * BI
    j0 = (wid % NJ) * BJ
    w0 = (S - 1) - (i0 + BI - 1) + j0
    src_top = (2 * S - 1) - w0

    for t in range(WROWS):
        pltpu.make_async_copy(
            relp_hbm.at[pl.ds((src_top - t) * D, D)],
            win_v.at[pl.ds(t * D, D)], sem_r).start()
    for _ in range(WROWS):
        pltpu.make_async_copy(
            relp_hbm.at[pl.ds(0, D)], win_v.at[pl.ds(0, D)], sem_r).wait()

    for r in range(BI):
        pltpu.make_async_copy(
            win_v.at[pl.ds((BI - 1 - r) * D, BJ * D)],
            out_hbm.at[pl.ds(((i0 + r) * S + j0) * D, BJ * D)], sem_w).start()
    for _ in range(BI):
        pltpu.make_async_copy(
            win_v.at[pl.ds(0, BJ * D)],
            out_hbm.at[pl.ds(0, BJ * D)], sem_w).wait()


def kernel(seq_len, rel_pos_emb):
    del seq_len
    max_len = (rel_pos_emb.shape[0] + 1) // 2
    start = max_len - 1 - (S - 1)
    relp = lax.slice_in_dim(rel_pos_emb, start - 1, start + 2 * S - 1, axis=0)
    out_flat = _expand(relp.reshape(2 * S * D))
    return out_flat.reshape(S, S, D)


# speedup vs baseline: 3.6241x; 3.1863x over previous
import functools

import jax
import jax.numpy as jnp
from jax import lax
from jax.experimental import pallas as pl
from jax.experimental.pallas import tpu as pltpu
from jax.experimental.pallas import tpu_sc as plsc

S = 512
D = 512
NC = 2
NS = 16
NW = NC * NS
T = 16
BJ = 128
WROWS = 8 * (T - 1) + BJ
NITEM = 4

_mesh = plsc.VectorSubcoreMesh(core_axis_name="c", subcore_axis_name="s")


@functools.partial(
    pl.kernel,
    mesh=_mesh,
    out_type=jax.ShapeDtypeStruct((S * S, D), jnp.float32),
    scratch_types=[
        pltpu.VMEM((2 * T * 16,), jnp.int32),
        pltpu.VMEM((WROWS, D), jnp.float32),
        pltpu.SemaphoreType.DMA,
        pltpu.SemaphoreType.DMA,
    ],
)
def _expand(relp_hbm, out_hbm, idx_v, win_v, sem_r, sem_w):
    wid = lax.axis_index("s") * NC + lax.axis_index("c")
    lane = lax.broadcasted_iota(jnp.int32, (16,), 0)

    for cc in range(NITEM):
        combo = wid * NITEM + cc
        m = combo % 8
        p = (combo // 8) % 4
        jb = combo // 32
        ibase = m + 128 * p
        j0 = jb * BJ
        w0 = (S - 1) - (ibase + 8 * (T - 1)) + j0
        top = (2 * S - 1) - w0
        for u in range(WROWS // 16 + 1):
            idx_v[pl.ds(u * 16, 16)] = (top - u * 16) - lane
        pltpu.make_async_copy(
            relp_hbm.at[idx_v.at[pl.ds(0, 128)]],
            win_v.at[pl.ds(0, 128)], sem_r).start()
        pltpu.make_async_copy(
            relp_hbm.at[idx_v.at[pl.ds(128, WROWS - 128)]],
            win_v.at[pl.ds(128, WROWS - 128)], sem_r).start()
        pltpu.make_async_copy(
            relp_hbm.at[idx_v.at[pl.ds(0, 128)]],
            win_v.at[pl.ds(0, 128)], sem_r).wait()
        pltpu.make_async_copy(
            relp_hbm.at[idx_v.at[pl.ds(128, WROWS - 128)]],
            win_v.at[pl.ds(128, WROWS - 128)], sem_r).wait()

        for t in range(T):
            pltpu.make_async_copy(
                win_v.at[pl.ds(8 * (T - 1 - t), BJ)],
                out_hbm.at[pl.ds((ibase + 8 * t) * S + j0, BJ)],
                sem_w).start()
        for _ in range(T):
            pltpu.make_async_copy(
                win_v.at[pl.ds(0, BJ)],
                out_hbm.at[pl.ds(0, BJ)], sem_w).wait()


def kernel(seq_len, rel_pos_emb):
    del seq_len
    max_len = (rel_pos_emb.shape[0] + 1) // 2
    start = max_len - 1 - (S - 1)
    relp = lax.slice_in_dim(rel_pos_emb, start - 1, start + 2 * S - 1, axis=0)
    out_flat = _expand(relp)
    return out_flat.reshape(S, S, D)
